# baseline (speedup 1.0000x reference)
import functools

import jax
import jax.numpy as jnp
from jax import lax
from jax.experimental import pallas as pl
from jax.experimental.pallas import tpu as pltpu
from jax.experimental.pallas import tpu_sc as plsc

B = 16384
D = 32
NC = 2
NS = 16
NW = NC * NS
BPW = B // NW
CHUNK = 128
NCHUNK = BPW // CHUNK
LANES = 16
NBLK = BPW // LANES


_mesh = plsc.VectorSubcoreMesh(core_axis_name="c", subcore_axis_name="s")


@functools.partial(
    pl.kernel,
    mesh=_mesh,
    compiler_params=pltpu.CompilerParams(
        use_tc_tiling_on_sc=False, needs_layout_passes=False
    ),
    out_type=jax.ShapeDtypeStruct((NW, BPW), jnp.float32),
    scratch_types=[
        pltpu.VMEM((NCHUNK, CHUNK), jnp.int32),
        pltpu.VMEM((NCHUNK, CHUNK), jnp.int32),
        pltpu.VMEM((NCHUNK, CHUNK), jnp.int32),
        pltpu.VMEM((BPW, D), jnp.float32),
        pltpu.VMEM((BPW, D), jnp.float32),
        pltpu.VMEM((BPW, D), jnp.float32),
        pltpu.VMEM((BPW,), jnp.float32),
        pltpu.SemaphoreType.DMA,
    ],
)
def _bpr_scores_sc(uidx_hbm, pidx_hbm, nidx_hbm, item_hbm, user_hbm,
                   out_hbm, uidx_v, pidx_v, nidx_v, urows, prows, nrows,
                   diffs, sem):
    wid = lax.axis_index("s") * NC + lax.axis_index("c")

    pltpu.sync_copy(uidx_hbm.at[wid], uidx_v)
    pltpu.sync_copy(pidx_hbm.at[wid], pidx_v)
    pltpu.sync_copy(nidx_hbm.at[wid], nidx_v)

    copies = []
    for j in range(NCHUNK):
        sl = pl.ds(j * CHUNK, CHUNK)
        copies.append(pltpu.async_copy(user_hbm.at[uidx_v.at[j]], urows.at[sl], sem))
        copies.append(pltpu.async_copy(item_hbm.at[pidx_v.at[j]], prows.at[sl], sem))
        copies.append(pltpu.async_copy(item_hbm.at[nidx_v.at[j]], nrows.at[sl], sem))
    for cp in copies:
        cp.wait()

    def block(i, carry):
        rid = i * LANES + lax.iota(jnp.int32, LANES)
        acc = jnp.zeros((LANES,), jnp.float32)
        for d in range(D):
            cid = jnp.full((LANES,), d, jnp.int32)
            u = plsc.load_gather(urows, [rid, cid])
            p = plsc.load_gather(prows, [rid, cid])
            n = plsc.load_gather(nrows, [rid, cid])
            acc = acc + u * (p - n)
        diffs[pl.ds(i * LANES, LANES)] = acc
        return carry

    lax.fori_loop(0, NBLK, block, 0)

    pltpu.sync_copy(diffs, out_hbm.at[wid])


def _loss_tc_body(x_ref, o_ref):
    x = x_ref[...]
    ls = jnp.minimum(x, 0.0) - jnp.log1p(jnp.exp(-jnp.abs(x)))
    o_ref[0, 0] = -jnp.sum(ls) / B


_loss_tc = pl.pallas_call(
    _loss_tc_body,
    out_shape=jax.ShapeDtypeStruct((1, 1), jnp.float32),
    out_specs=pl.BlockSpec(memory_space=pltpu.SMEM),
)


def kernel(user_seqs, next_items, neg_items, item_table, user_table):
    uidx = user_seqs.astype(jnp.int32).reshape(NW, NCHUNK, CHUNK)
    pidx = (next_items.astype(jnp.int32) - 1).reshape(NW, NCHUNK, CHUNK)
    nidx = (neg_items.astype(jnp.int32) - 1).reshape(NW, NCHUNK, CHUNK)
    diffs = _bpr_scores_sc(uidx, pidx, nidx, item_table, user_table)
    return _loss_tc(diffs.reshape(B // 128, 128))[0, 0]


# speedup vs baseline: 2.6120x; 2.6120x over previous
import functools

import jax
import jax.numpy as jnp
from jax import lax
from jax.experimental import pallas as pl
from jax.experimental.pallas import tpu as pltpu
from jax.experimental.pallas import tpu_sc as plsc

B = 16384
D = 32
NROW = 1000000
NC = 2
NS = 16
NW = NC * NS
BPW = B // NW
LANES = 16
SUBL = 8
DI = D // SUBL
CH = 4
NCHUNK = BPW // CH
RING = 2 * CH


_mesh = plsc.VectorSubcoreMesh(core_axis_name="c", subcore_axis_name="s")


@functools.partial(
    pl.kernel,
    mesh=_mesh,
    compiler_params=pltpu.CompilerParams(needs_layout_passes=False),
    out_type=jax.ShapeDtypeStruct((NW, BPW), jnp.float32),
    scratch_types=[
        pltpu.VMEM((BPW + LANES,), jnp.int32),
        pltpu.VMEM((BPW + LANES,), jnp.int32),
        pltpu.VMEM((BPW + LANES,), jnp.int32),
        pltpu.VMEM((DI * RING, SUBL, 128), jnp.float32),
        pltpu.VMEM((DI * RING, SUBL, 128), jnp.float32),
        pltpu.VMEM((DI * RING, SUBL, 128), jnp.float32),
        pltpu.VMEM((BPW,), jnp.float32),
        pltpu.SemaphoreType.DMA((2,)),
    ],
)
def _bpr_scores_sc(uidx_hbm, pidx_hbm, nidx_hbm, item_hbm, user_hbm,
                   out_hbm, uidx_v, pidx_v, nidx_v, ubuf, pbuf, nbuf,
                   diffs, sem):
    wid = lax.axis_index("s") * NC + lax.axis_index("c")

    pltpu.sync_copy(uidx_hbm.at[wid], uidx_v.at[pl.ds(0, BPW)])
    pltpu.sync_copy(pidx_hbm.at[wid], pidx_v.at[pl.ds(0, BPW)])
    pltpu.sync_copy(nidx_hbm.at[wid], nidx_v.at[pl.ds(0, BPW)])

    def zero(b, carry):
        diffs[pl.ds(b * LANES, LANES)] = jnp.zeros((LANES,), jnp.float32)
        return carry

    lax.fori_loop(0, BPW // LANES, zero, 0)

    def issue_chunk(c):
        par = lax.rem(c, 2)
        ru = uidx_v[pl.ds(c * CH, LANES)]
        rp = pidx_v[pl.ds(c * CH, LANES)]
        rn = nidx_v[pl.ds(c * CH, LANES)]
        for l in range(CH):
            slot = par * CH + l
            for r_vec, tbl, buf in ((ru, user_hbm, ubuf),
                                    (rp, item_hbm, pbuf),
                                    (rn, item_hbm, nbuf)):
                colbase = pl.multiple_of((r_vec[l] >> 7) << 7, 128)
                for i in range(DI):
                    pltpu.async_copy(
                        tbl.at[pl.ds(i, 1), :, pl.ds(colbase, 128)],
                        buf.at[pl.ds(i * RING + slot, 1), :, :],
                        sem.at[par])

    def drain_chunk(c):
        par = lax.rem(c, 2)

        def w(_, carry):
            pltpu.make_async_copy(
                user_hbm.at[pl.ds(0, 1), :, pl.ds(0, 128)],
                ubuf.at[pl.ds(0, 1), :, :], sem.at[par],
            ).wait()
            return carry

        lax.fori_loop(0, 3 * CH * DI, w, 0)

    dv = lax.iota(jnp.int32, LANES)
    jv = dv & 7
    iv_lo = (dv >> 3) * RING
    iv_hi = iv_lo + 2 * RING

    def compute_chunk(c):
        par = lax.rem(c, 2)
        ru = uidx_v[pl.ds(c * CH, LANES)]
        rp = pidx_v[pl.ds(c * CH, LANES)]
        rn = nidx_v[pl.ds(c * CH, LANES)]
        blk = (c * CH) // LANES
        contrib = jnp.zeros((LANES,), jnp.float32)
        for l in range(CH):
            slot = par * CH + l
            su = iv_lo + slot
            s_sum = jnp.float32(0.0)
            for half, ivb in ((0, iv_lo), (1, iv_hi)):
                lu = jnp.full((LANES,), ru[l] & 127, jnp.int32)
                lp = jnp.full((LANES,), rp[l] & 127, jnp.int32)
                ln = jnp.full((LANES,), rn[l] & 127, jnp.int32)
                u = plsc.load_gather(ubuf, [ivb + slot, jv, lu])
                pv = plsc.load_gather(pbuf, [ivb + slot, jv, lp])
                nv = plsc.load_gather(nbuf, [ivb + slot, jv, ln])
                s_sum = s_sum + jnp.sum(u * (pv - nv))
            pos = (c * CH + l) % LANES
            contrib = contrib + jnp.where(dv == pos, s_sum, 0.0)
        sl = pl.ds(blk * LANES, LANES)
        diffs[sl] = diffs[sl] + contrib

    def step(s, carry):
        @pl.when(s < NCHUNK)
        def _():
            issue_chunk(s)

        @pl.when(s > 0)
        def _():
            drain_chunk(s - 1)
            compute_chunk(s - 1)

        return carry

    lax.fori_loop(0, NCHUNK + 1, step, 0)

    pltpu.sync_copy(diffs, out_hbm.at[wid])


def _loss_tc_body(x_ref, o_ref):
    x = x_ref[...]
    ls = jnp.minimum(x, 0.0) - jnp.log1p(jnp.exp(-jnp.abs(x)))
    o_ref[0, 0] = -jnp.sum(ls) / B


_loss_tc = pl.pallas_call(
    _loss_tc_body,
    out_shape=jax.ShapeDtypeStruct((1, 1), jnp.float32),
    out_specs=pl.BlockSpec(memory_space=pltpu.SMEM),
)


def kernel(user_seqs, next_items, neg_items, item_table, user_table):
    uidx = user_seqs.astype(jnp.int32).reshape(NW, BPW)
    pidx = (next_items.astype(jnp.int32) - 1).reshape(NW, BPW)
    nidx = (neg_items.astype(jnp.int32) - 1).reshape(NW, BPW)
    item_v = item_table.T.reshape(DI, SUBL, NROW)
    user_v = user_table.T.reshape(DI, SUBL, NROW)
    diffs = _bpr_scores_sc(uidx, pidx, nidx, item_v, user_v)
    return _loss_tc(diffs.reshape(B // 128, 128))[0, 0]


# speedup vs baseline: 2.6252x; 1.0050x over previous
import functools

import jax
import jax.numpy as jnp
from jax import lax
from jax.experimental import pallas as pl
from jax.experimental.pallas import tpu as pltpu
from jax.experimental.pallas import tpu_sc as plsc

B = 16384
D = 32
NROW = 1000000
NC = 2
NS = 16
NW = NC * NS
BPW = B // NW
LANES = 16
SUBL = 8
DI = D // SUBL
CH = 4
NCHUNK = BPW // CH
RING = 2 * CH


_mesh = plsc.VectorSubcoreMesh(core_axis_name="c", subcore_axis_name="s")


@functools.partial(
    pl.kernel,
    mesh=_mesh,
    compiler_params=pltpu.CompilerParams(needs_layout_passes=False),
    out_type=jax.ShapeDtypeStruct((NW, BPW), jnp.float32),
    scratch_types=[
        pltpu.VMEM((BPW + LANES,), jnp.int32),
        pltpu.VMEM((BPW + LANES,), jnp.int32),
        pltpu.VMEM((BPW + LANES,), jnp.int32),
        pltpu.VMEM((DI * RING, SUBL, 128), jnp.float32),
        pltpu.VMEM((DI * RING, SUBL, 128), jnp.float32),
        pltpu.VMEM((DI * RING, SUBL, 128), jnp.float32),
        pltpu.VMEM((BPW,), jnp.float32),
        pltpu.SemaphoreType.DMA((2,)),
    ],
)
def _bpr_scores_sc(uidx_hbm, pidx_hbm, nidx_hbm, item_hbm, user_hbm,
                   out_hbm, uidx_v, pidx_v, nidx_v, ubuf, pbuf, nbuf,
                   diffs, sem):
    wid = lax.axis_index("s") * NC + lax.axis_index("c")

    pltpu.sync_copy(uidx_hbm.at[wid], uidx_v.at[pl.ds(0, BPW)])
    pltpu.sync_copy(pidx_hbm.at[wid], pidx_v.at[pl.ds(0, BPW)])
    pltpu.sync_copy(nidx_hbm.at[wid], nidx_v.at[pl.ds(0, BPW)])

    def zero(b, carry):
        diffs[pl.ds(b * LANES, LANES)] = jnp.zeros((LANES,), jnp.float32)
        return carry

    lax.fori_loop(0, BPW // LANES, zero, 0)

    def issue_chunk(c):
        par = lax.rem(c, 2)
        ru = uidx_v[pl.ds(c * CH, LANES)]
        rp = pidx_v[pl.ds(c * CH, LANES)]
        rn = nidx_v[pl.ds(c * CH, LANES)]
        for l in range(CH):
            slot = par * CH + l
            for r_vec, tbl, buf in ((ru, user_hbm, ubuf),
                                    (rp, item_hbm, pbuf),
                                    (rn, item_hbm, nbuf)):
                colbase = pl.multiple_of((r_vec[l] >> 7) << 7, 128)
                for i in range(DI):
                    pltpu.async_copy(
                        tbl.at[pl.ds(i, 1), :, pl.ds(colbase, 128)],
                        buf.at[pl.ds(i * RING + slot, 1), :, :],
                        sem.at[par])

    def drain_chunk(c):
        par = lax.rem(c, 2)

        def w(_, carry):
            pltpu.make_async_copy(
                user_hbm.at[pl.ds(0, 1), :, pl.ds(0, 128)],
                ubuf.at[pl.ds(0, 1), :, :], sem.at[par],
            ).wait()
            return carry

        lax.fori_loop(0, 3 * CH * DI, w, 0)

    dv = lax.iota(jnp.int32, LANES)
    jv = dv & 7
    iv_lo = (dv >> 3) * RING
    iv_hi = iv_lo + 2 * RING

    def compute_chunk(c):
        par = lax.rem(c, 2)
        ru = uidx_v[pl.ds(c * CH, LANES)]
        rp = pidx_v[pl.ds(c * CH, LANES)]
        rn = nidx_v[pl.ds(c * CH, LANES)]
        blk = (c * CH) // LANES
        contrib = jnp.zeros((LANES,), jnp.float32)
        for l in range(CH):
            slot = par * CH + l
            s_sum = jnp.float32(0.0)
            for half, ivb in ((0, iv_lo), (1, iv_hi)):
                lu = jnp.full((LANES,), ru[l] & 127, jnp.int32)
                lp = jnp.full((LANES,), rp[l] & 127, jnp.int32)
                ln = jnp.full((LANES,), rn[l] & 127, jnp.int32)
                u = plsc.load_gather(ubuf, [ivb + slot, jv, lu])
                pv = plsc.load_gather(pbuf, [ivb + slot, jv, lp])
                nv = plsc.load_gather(nbuf, [ivb + slot, jv, ln])
                s_sum = s_sum + jnp.sum(u * (pv - nv))
            pos = (c * CH + l) % LANES
            contrib = contrib + jnp.where(dv == pos, s_sum, 0.0)
        sl = pl.ds(blk * LANES, LANES)
        diffs[sl] = diffs[sl] + contrib

    def step(s, carry):
        @pl.when(s < NCHUNK)
        def _():
            issue_chunk(s)

        @pl.when(s > 0)
        def _():
            drain_chunk(s - 1)
            compute_chunk(s - 1)

        return carry

    lax.fori_loop(0, NCHUNK + 1, step, 0)

    pltpu.sync_copy(diffs, out_hbm.at[wid])


def _loss_tc_body(x_ref, o_ref):
    x = x_ref[...]
    ls = jnp.minimum(x, 0.0) - jnp.log1p(jnp.exp(-jnp.abs(x)))
    o_ref[0, 0] = -jnp.sum(ls) / B


_loss_tc = pl.pallas_call(
    _loss_tc_body,
    out_shape=jax.ShapeDtypeStruct((1, 1), jnp.float32),
    out_specs=pl.BlockSpec(memory_space=pltpu.SMEM),
)


def kernel(user_seqs, next_items, neg_items, item_table, user_table):
    uidx = user_seqs.astype(jnp.int32).reshape(NW, BPW)
    pidx = (next_items.astype(jnp.int32) - 1).reshape(NW, BPW)
    nidx = (neg_items.astype(jnp.int32) - 1).reshape(NW, BPW)
    item_v = item_table.T.reshape(DI, SUBL, NROW)
    user_v = user_table.T.reshape(DI, SUBL, NROW)
    diffs = _bpr_scores_sc(uidx, pidx, nidx, item_v, user_v)
    return _loss_tc(diffs.reshape(B // 128, 128))[0, 0]
